# baseline (speedup 1.0000x reference)
import jax
import jax.numpy as jnp
from jax.experimental import pallas as pl
from jax.experimental.pallas import tpu as pltpu

E = 8
C_IN = 256
C_OUT = 256
K = 5
G = 8
B = 64
L = 2048
EPS = 1e-5
PB = 2


def _conv(xslab, w):
    xp = jnp.pad(xslab.astype(jnp.bfloat16),
                 ((0, 0), (K // 2, K // 2)))
    xs = jnp.concatenate([xp[:, k:k + L] for k in range(K)], axis=0)
    return jax.lax.dot_general(
        w, xs, (((1,), (0,)), ((), ())),
        preferred_element_type=jnp.float32)


def _stats(acc, bias, gamma, beta, m):
    n = (C_OUT // G) * L
    s1 = jnp.sum(acc, axis=1, keepdims=True)
    s2 = jnp.sum(acc * acc, axis=1, keepdims=True)
    s2 = s2 + (2.0 * bias) * s1 + (L * 1.0) * bias * bias
    s1 = s1 + L * bias
    gs = jax.lax.dot_general(
        m, jnp.concatenate([s1, s2], axis=1),
        (((1,), (0,)), ((), ())),
        preferred_element_type=jnp.float32)
    mu_c = gs[:, 0:1] / n
    var_c = gs[:, 1:2] / n - mu_c * mu_c
    r_c = jax.lax.rsqrt(var_c + EPS)
    scale = r_c * gamma
    shift = (bias - mu_c) * scale + beta
    return scale, shift


def _finish(acc, scale, shift):
    y = acc * scale + shift
    u = jnp.exp(jnp.minimum(y, 30.0))
    num = u * (u + 2.0)
    return y * (num / (num + 2.0))


def _body(idx_ref, x_ref, *refs):
    w_refs = refs[:PB]
    b_refs = refs[PB:2 * PB]
    g_refs = refs[2 * PB:3 * PB]
    bt_refs = refs[3 * PB:4 * PB]
    o_ref = refs[4 * PB]
    m_ref = refs[4 * PB + 1]

    @pl.when(pl.program_id(0) == 0)
    def _build_mask():
        cpg = C_OUT // G
        gi = jax.lax.broadcasted_iota(jnp.int32, (C_OUT, C_OUT), 0) // cpg
        gj = jax.lax.broadcasted_iota(jnp.int32, (C_OUT, C_OUT), 1) // cpg
        m_ref[...] = (gi == gj).astype(jnp.float32)

    def _col(r):
        return r[0, 0].reshape(C_OUT, 1)

    def _post(j):
        sc, sh = _stats(acc[j], _col(b_refs[j]), _col(g_refs[j]),
                        _col(bt_refs[j]), m_ref[...])
        o_ref[j] = _finish(acc[j], sc, sh)

    acc = [None] * PB
    for j in range(PB):
        acc[j] = _conv(x_ref[j], w_refs[j][0])
        if j > 0:
            _post(j - 1)
    _post(PB - 1)


def kernel(x, use_expert_i, conv_w, conv_b, gn_gamma, gn_beta):
    wt = (jnp.transpose(conv_w, (0, 1, 3, 2))
          .reshape(E, C_OUT, K * C_IN).astype(jnp.bfloat16))
    b3 = conv_b.reshape(E, 1, C_OUT)
    g3 = gn_gamma.reshape(E, 1, C_OUT)
    bt3 = gn_beta.reshape(E, 1, C_OUT)

    w_specs = [
        pl.BlockSpec((1, C_OUT, K * C_IN),
                     lambda i, idx, j=j: (idx[PB * i + j], 0, 0))
        for j in range(PB)
    ]
    p_specs = [
        pl.BlockSpec((1, 1, C_OUT), lambda i, idx, j=j: (idx[PB * i + j], 0, 0))
        for j in range(PB)
    ]
    grid_spec = pltpu.PrefetchScalarGridSpec(
        num_scalar_prefetch=1,
        grid=(B // PB,),
        in_specs=(
            [pl.BlockSpec((PB, C_IN, L), lambda i, idx: (i, 0, 0))]
            + w_specs + p_specs + p_specs + p_specs
        ),
        out_specs=pl.BlockSpec((PB, C_OUT, L), lambda i, idx: (i, 0, 0)),
        scratch_shapes=[pltpu.VMEM((C_OUT, C_OUT), jnp.float32)],
    )
    return pl.pallas_call(
        _body,
        grid_spec=grid_spec,
        out_shape=jax.ShapeDtypeStruct((B, C_OUT, L), jnp.float32),
        compiler_params=pltpu.CompilerParams(
            dimension_semantics=("parallel",),
        ),
    )(use_expert_i, x, *([wt] * PB), *([b3] * PB), *([g3] * PB),
      *([bt3] * PB))


# speedup vs baseline: 1.0091x; 1.0091x over previous
import jax
import jax.numpy as jnp
from jax.experimental import pallas as pl
from jax.experimental.pallas import tpu as pltpu

E = 8
C_IN = 256
C_OUT = 256
K = 5
G = 8
B = 64
L = 2048
EPS = 1e-5
PB = 2


def _conv(xslab, w):
    xp = jnp.pad(xslab.astype(jnp.bfloat16),
                 ((0, 0), (K // 2, K // 2)))
    xs = jnp.concatenate([xp[:, k:k + L] for k in range(K)], axis=0)
    return jax.lax.dot_general(
        w, xs, (((1,), (0,)), ((), ())),
        preferred_element_type=jnp.float32)


def _stats(acc, p, m):
    n = (C_OUT // G) * L
    bias = p[0].reshape(C_OUT, 1)
    s1 = jnp.sum(acc, axis=1, keepdims=True)
    s2 = jnp.sum(acc * acc, axis=1, keepdims=True)
    s2 = s2 + (2.0 * bias) * s1 + (L * 1.0) * bias * bias
    s1 = s1 + L * bias
    gs = jax.lax.dot_general(
        m, jnp.concatenate([s1, s2], axis=1),
        (((1,), (0,)), ((), ())),
        preferred_element_type=jnp.float32)
    mu_c = gs[:, 0:1] / n
    var_c = gs[:, 1:2] / n - mu_c * mu_c
    r_c = jax.lax.rsqrt(var_c + EPS)
    scale = r_c * p[1].reshape(C_OUT, 1)
    shift = (bias - mu_c) * scale + p[2].reshape(C_OUT, 1)
    return scale, shift


def _finish(acc, scale, shift):
    y = acc * scale + shift
    u = jnp.exp(jnp.minimum(y, 30.0))
    num = u * (u + 2.0)
    return y * (num / (num + 2.0))


def _body(idx_ref, x_ref, *refs):
    w_refs = refs[:PB]
    p_refs = refs[PB:2 * PB]
    m_ref = refs[2 * PB]
    o_ref = refs[2 * PB + 1]
    acc = [None] * PB
    for j in range(PB):
        acc[j] = _conv(x_ref[j], w_refs[j][0])
        if j > 0:
            sc, sh = _stats(acc[j - 1], p_refs[j - 1][0], m_ref[...])
            o_ref[j - 1] = _finish(acc[j - 1], sc, sh)
    sc, sh = _stats(acc[PB - 1], p_refs[PB - 1][0], m_ref[...])
    o_ref[PB - 1] = _finish(acc[PB - 1], sc, sh)


def kernel(x, use_expert_i, conv_w, conv_b, gn_gamma, gn_beta):
    wt = (jnp.transpose(conv_w, (0, 1, 3, 2))
          .reshape(E, C_OUT, K * C_IN).astype(jnp.bfloat16))
    params = jnp.stack([conv_b, gn_gamma, gn_beta], axis=1)
    cpg = C_OUT // G
    gi = jnp.arange(C_OUT, dtype=jnp.int32) // cpg
    gmask = (gi[:, None] == gi[None, :]).astype(jnp.float32)

    w_specs = [
        pl.BlockSpec((1, C_OUT, K * C_IN),
                     lambda i, idx, j=j: (idx[PB * i + j], 0, 0))
        for j in range(PB)
    ]
    p_specs = [
        pl.BlockSpec((1, 3, C_OUT), lambda i, idx, j=j: (idx[PB * i + j], 0, 0))
        for j in range(PB)
    ]
    grid_spec = pltpu.PrefetchScalarGridSpec(
        num_scalar_prefetch=1,
        grid=(B // PB,),
        in_specs=(
            [pl.BlockSpec((PB, C_IN, L), lambda i, idx: (i, 0, 0))]
            + w_specs + p_specs
            + [pl.BlockSpec((C_OUT, C_OUT), lambda i, idx: (0, 0))]
        ),
        out_specs=pl.BlockSpec((PB, C_OUT, L), lambda i, idx: (i, 0, 0)),
    )
    return pl.pallas_call(
        _body,
        grid_spec=grid_spec,
        out_shape=jax.ShapeDtypeStruct((B, C_OUT, L), jnp.float32),
        compiler_params=pltpu.CompilerParams(
            dimension_semantics=("parallel",),
        ),
    )(use_expert_i, x, *([wt] * PB), *([params] * PB), gmask)
